# baseline (speedup 1.0000x reference)
import jax
import jax.numpy as jnp
from jax import lax
from jax.experimental import pallas as pl
from jax.experimental.pallas import tpu as pltpu
from jax.experimental.pallas import tpu_sc as plsc

N = 10000
CH = 128
NPAD = 10240
DUMMY = NPAD - 1
NCORES = 2
NSUB = 16
NTILES = NCORES * NSUB
CHUNK = 128
NCHUNK = 80
EPT = NCHUNK * CHUNK
EPAD = NTILES * EPT
RPT = NPAD // NSUB
BLK = 1024


def _mesh():
    return plsc.VectorSubcoreMesh(core_axis_name="c", subcore_axis_name="s")


CPT = EPAD // NSUB // CHUNK
CHALF = CPT // 2


def _deg_body(cols_hbm, deg_out, colv, pdeg, tbuf, rbuf, repv, sh):
    c = lax.axis_index("c")
    s = lax.axis_index("s")

    def z(i, _):
        pdeg[pl.ds(i * 16, 16)] = jnp.zeros((16,), jnp.float32)
        return 0

    lax.fori_loop(0, NPAD // 16, z, 0)

    ones = jnp.ones((16,), jnp.float32)
    for h in range(2):
        pltpu.sync_copy(cols_hbm.at[s, pl.ds(h * CHALF, CHALF)], colv)

        def count(j, _):
            for u in range(CHUNK // 16):
                iv = colv[j, pl.ds(u * 16, 16)]
                plsc.addupdate_scatter(pdeg, [iv], ones)
            return 0

        lax.fori_loop(0, CHALF, count, 0)

    pltpu.sync_copy(pdeg, sh.at[s])
    plsc.subcore_barrier()
    pltpu.sync_copy(sh.at[:, pl.ds(s * RPT, RPT)], tbuf)

    def red(k, _):
        v = tbuf[0, pl.ds(k * 16, 16)]
        for t in range(1, NSUB):
            v = v + tbuf[t, pl.ds(k * 16, 16)]
        rbuf[pl.ds(k * 16, 16)] = v
        return 0

    lax.fori_loop(0, RPT // 16, red, 0)

    @pl.when(c == 0)
    def _():
        for k in range(RPT // CHUNK):
            def rep(r, _):
                iv = jnp.full((16,), k * CHUNK + r, jnp.int32)
                d = plsc.load_gather(rbuf, [iv])
                for u in range(CH // 16):
                    repv[r, pl.ds(u * 16, 16)] = d
                return 0

            lax.fori_loop(0, CHUNK, rep, 0)
            pltpu.sync_copy(repv,
                            deg_out.at[pl.ds(s * RPT + k * CHUNK, CHUNK)])


_deg_call = pl.kernel(
    _deg_body,
    out_type=jax.ShapeDtypeStruct((NPAD, CH), jnp.float32),
    mesh=_mesh(),
    compiler_params=pltpu.CompilerParams(needs_layout_passes=False),
    scratch_types=[
        pltpu.VMEM((CHALF, CHUNK), jnp.int32),
        pltpu.VMEM((NPAD,), jnp.float32),
        pltpu.VMEM((NSUB, RPT), jnp.float32),
        pltpu.VMEM((RPT,), jnp.float32),
        pltpu.VMEM((CHUNK, CH), jnp.float32),
        pltpu.VMEM_SHARED((NSUB, NPAD), jnp.float32),
    ],
)


HALF = NCHUNK // 2


def _msg_body(rows_hbm, cols_hbm, y_hbm, zeros_hbm, parts_out, rowv, colv,
              bufa, bufb, acc, gsa, gsb, ssa, ssb):
    c = lax.axis_index("c")
    s = lax.axis_index("s")
    wid = s * NCORES + c

    pltpu.sync_copy(zeros_hbm, acc.at[pl.ds(s * RPT, RPT)])
    plsc.subcore_barrier()

    def run_slice(wid):
      for h in range(NCHUNK // HALF):
        pltpu.sync_copy(rows_hbm.at[wid, pl.ds(h * HALF, HALF)], rowv)
        pltpu.sync_copy(cols_hbm.at[wid, pl.ds(h * HALF, HALF)], colv)
        pltpu.async_copy(y_hbm.at[rowv.at[0]], bufa, gsa).wait()

        def pair(p, _):
            j = p * 2
            sh = pltpu.async_copy(bufa, acc.at[colv.at[j]], ssa, add=True)
            gh = pltpu.async_copy(y_hbm.at[rowv.at[j + 1]], bufb, gsb)
            gh.wait()
            sh.wait()
            sh2 = pltpu.async_copy(bufb, acc.at[colv.at[j + 1]], ssb, add=True)
            gh2 = pltpu.async_copy(y_hbm.at[rowv.at[j + 2]], bufa, gsa)
            gh2.wait()
            sh2.wait()
            return 0

        lax.fori_loop(0, (HALF - 2) // 2, pair, 0)
        sh = pltpu.async_copy(bufa, acc.at[colv.at[HALF - 2]], ssa, add=True)
        gh = pltpu.async_copy(y_hbm.at[rowv.at[HALF - 1]], bufb, gsb)
        gh.wait()
        sh.wait()
        pltpu.async_copy(bufb, acc.at[colv.at[HALF - 1]], ssb, add=True).wait()

    @pl.when(c == 0)
    def _():
        run_slice(s * NCORES)
        run_slice(s * NCORES + 1)

    plsc.subcore_barrier()
    pltpu.sync_copy(
        acc.at[pl.ds(s * RPT, RPT)], parts_out.at[c, pl.ds(s * RPT, RPT)]
    )


_msg_call = pl.kernel(
    _msg_body,
    out_type=jax.ShapeDtypeStruct((NCORES, NPAD, CH), jnp.float32),
    mesh=_mesh(),
    scratch_types=[
        pltpu.VMEM((HALF, CHUNK), jnp.int32),
        pltpu.VMEM((HALF, CHUNK), jnp.int32),
        pltpu.VMEM((CHUNK, CH), jnp.float32),
        pltpu.VMEM((CHUNK, CH), jnp.float32),
        pltpu.VMEM_SHARED((NPAD, CH), jnp.float32),
        pltpu.SemaphoreType.DMA,
        pltpu.SemaphoreType.DMA,
        pltpu.SemaphoreType.DMA,
        pltpu.SemaphoreType.DMA,
    ],
)


def _pre_body(x_ref, w_ref, deg_ref, y_ref, dis_ref):
    xw = jnp.dot(
        x_ref[...], w_ref[...],
        preferred_element_type=jnp.float32,
        precision=lax.Precision.HIGHEST,
    )
    dis = lax.rsqrt(deg_ref[...] + 1.0)
    y_ref[...] = xw * dis
    dis_ref[...] = dis


def _pre_call(x_pad, w, deg_rep):
    return pl.pallas_call(
        _pre_body,
        grid=(NPAD // BLK,),
        in_specs=[
            pl.BlockSpec((BLK, CH), lambda i: (i, 0)),
            pl.BlockSpec((CH, CH), lambda i: (0, 0)),
            pl.BlockSpec((BLK, CH), lambda i: (i, 0)),
        ],
        out_specs=[
            pl.BlockSpec((BLK, CH), lambda i: (i, 0)),
            pl.BlockSpec((BLK, CH), lambda i: (i, 0)),
        ],
        out_shape=[
            jax.ShapeDtypeStruct((NPAD, CH), jnp.float32),
            jax.ShapeDtypeStruct((NPAD, CH), jnp.float32),
        ],
    )(x_pad, w, deg_rep)


def _post_body(p_ref, y_ref, dis_ref, b_ref, a_ref, o_ref):
    si = p_ref[0, :, :] + p_ref[1, :, :] + y_ref[...]
    o = dis_ref[...] * si + b_ref[...]
    o_ref[...] = jnp.where(o > 0, o, a_ref[...] * o)


def _post_call(parts, y, dis, b2, a2):
    return pl.pallas_call(
        _post_body,
        grid=(NPAD // BLK,),
        in_specs=[
            pl.BlockSpec((NCORES, BLK, CH), lambda i: (0, i, 0)),
            pl.BlockSpec((BLK, CH), lambda i: (i, 0)),
            pl.BlockSpec((BLK, CH), lambda i: (i, 0)),
            pl.BlockSpec((1, CH), lambda i: (0, 0)),
            pl.BlockSpec((1, CH), lambda i: (0, 0)),
        ],
        out_specs=pl.BlockSpec((BLK, CH), lambda i: (i, 0)),
        out_shape=jax.ShapeDtypeStruct((NPAD, CH), jnp.float32),
    )(parts, y, dis, b2, a2)


def kernel(x, edge_index, W, b, prelu_a):
    row = edge_index[0]
    col = edge_index[1]
    e = row.shape[0]
    pad = EPAD - e
    rows = jnp.concatenate(
        [row, jnp.zeros((pad,), jnp.int32)]
    ).reshape(NTILES, NCHUNK, CHUNK)
    cols = jnp.concatenate(
        [col, jnp.full((pad,), DUMMY, jnp.int32)]
    ).reshape(NTILES, NCHUNK, CHUNK)
    x_pad = jnp.pad(x, ((0, NPAD - N), (0, 0)))

    zeros_ch = jnp.zeros((RPT, CH), jnp.float32)
    deg_rep = _deg_call(cols.reshape(NSUB, CPT, CHUNK))
    y, dis = _pre_call(x_pad, W, deg_rep)
    parts = _msg_call(rows, cols, y, zeros_ch)
    out = _post_call(parts, y, dis, b.reshape(1, CH), prelu_a.reshape(1, CH))
    return out[:N]


# speedup vs baseline: 1.1237x; 1.1237x over previous
import jax
import jax.numpy as jnp
from jax import lax
from jax.experimental import pallas as pl
from jax.experimental.pallas import tpu as pltpu
from jax.experimental.pallas import tpu_sc as plsc

N = 10000
CH = 128
NPAD = 10240
DUMMY = NPAD - 1
NCORES = 2
NSUB = 16
NTILES = NCORES * NSUB
CHUNK = 128
NCHUNK = 80
EPT = NCHUNK * CHUNK
EPAD = NTILES * EPT
RPT = NPAD // NSUB
BLK = 1024


def _mesh():
    return plsc.VectorSubcoreMesh(core_axis_name="c", subcore_axis_name="s")


CPT = EPAD // NSUB // CHUNK
CHALF = CPT // 2


def _deg_body(cols_hbm, deg_out, colv, pdeg, tbuf, rbuf, repv, sh):
    c = lax.axis_index("c")
    s = lax.axis_index("s")

    def z(i, _):
        pdeg[pl.ds(i * 16, 16)] = jnp.zeros((16,), jnp.float32)
        return 0

    lax.fori_loop(0, NPAD // 16, z, 0)

    ones = jnp.ones((16,), jnp.float32)
    for h in range(2):
        pltpu.sync_copy(cols_hbm.at[s, pl.ds(h * CHALF, CHALF)], colv)

        def count(j, _):
            for u in range(CHUNK // 16):
                iv = colv[j, pl.ds(u * 16, 16)]
                plsc.addupdate_scatter(pdeg, [iv], ones)
            return 0

        lax.fori_loop(0, CHALF, count, 0)

    pltpu.sync_copy(pdeg, sh.at[s])
    plsc.subcore_barrier()
    pltpu.sync_copy(sh.at[:, pl.ds(s * RPT, RPT)], tbuf)

    def red(k, _):
        v = tbuf[0, pl.ds(k * 16, 16)]
        for t in range(1, NSUB):
            v = v + tbuf[t, pl.ds(k * 16, 16)]
        rbuf[pl.ds(k * 16, 16)] = v
        return 0

    lax.fori_loop(0, RPT // 16, red, 0)

    @pl.when(c == 0)
    def _():
        for k in range(RPT // CHUNK):
            def rep(r, _):
                iv = jnp.full((16,), k * CHUNK + r, jnp.int32)
                d = plsc.load_gather(rbuf, [iv])
                for u in range(CH // 16):
                    repv[r, pl.ds(u * 16, 16)] = d
                return 0

            lax.fori_loop(0, CHUNK, rep, 0)
            pltpu.sync_copy(repv,
                            deg_out.at[pl.ds(s * RPT + k * CHUNK, CHUNK)])


_deg_call = pl.kernel(
    _deg_body,
    out_type=jax.ShapeDtypeStruct((NPAD, CH), jnp.float32),
    mesh=_mesh(),
    compiler_params=pltpu.CompilerParams(needs_layout_passes=False),
    scratch_types=[
        pltpu.VMEM((CHALF, CHUNK), jnp.int32),
        pltpu.VMEM((NPAD,), jnp.float32),
        pltpu.VMEM((NSUB, RPT), jnp.float32),
        pltpu.VMEM((RPT,), jnp.float32),
        pltpu.VMEM((CHUNK, CH), jnp.float32),
        pltpu.VMEM_SHARED((NSUB, NPAD), jnp.float32),
    ],
)


HALF = NCHUNK // 2


def _msg_body(rows_hbm, cols_hbm, y_hbm, parts_out, rowv, colv,
              bufa, bufb, acc, gsa, gsb, ssa, ssb):
    c = lax.axis_index("c")
    s = lax.axis_index("s")
    wid = s * NCORES + c

    def zfill(i, _):
        for u in range(CH // 16):
            bufa[i, pl.ds(u * 16, 16)] = jnp.zeros((16,), jnp.float32)
        return 0

    lax.fori_loop(0, CHUNK, zfill, 0)
    for k in range(RPT // CHUNK):
        pltpu.sync_copy(bufa, acc.at[pl.ds(s * RPT + k * CHUNK, CHUNK)])
    plsc.subcore_barrier()

    def run_slice(wid):
      for h in range(NCHUNK // HALF):
        pltpu.sync_copy(rows_hbm.at[wid, pl.ds(h * HALF, HALF)], rowv)
        pltpu.sync_copy(cols_hbm.at[wid, pl.ds(h * HALF, HALF)], colv)
        pltpu.async_copy(y_hbm.at[rowv.at[0]], bufa, gsa).wait()

        def pair(p, _):
            j = p * 2
            sh = pltpu.async_copy(bufa, acc.at[colv.at[j]], ssa, add=True)
            gh = pltpu.async_copy(y_hbm.at[rowv.at[j + 1]], bufb, gsb)
            gh.wait()
            sh.wait()
            sh2 = pltpu.async_copy(bufb, acc.at[colv.at[j + 1]], ssb, add=True)
            gh2 = pltpu.async_copy(y_hbm.at[rowv.at[j + 2]], bufa, gsa)
            gh2.wait()
            sh2.wait()
            return 0

        lax.fori_loop(0, (HALF - 2) // 2, pair, 0)
        sh = pltpu.async_copy(bufa, acc.at[colv.at[HALF - 2]], ssa, add=True)
        gh = pltpu.async_copy(y_hbm.at[rowv.at[HALF - 1]], bufb, gsb)
        gh.wait()
        sh.wait()
        pltpu.async_copy(bufb, acc.at[colv.at[HALF - 1]], ssb, add=True).wait()

    run_slice(wid)
    plsc.subcore_barrier()
    pltpu.sync_copy(
        acc.at[pl.ds(s * RPT, RPT)], parts_out.at[c, pl.ds(s * RPT, RPT)]
    )


_msg_call = pl.kernel(
    _msg_body,
    out_type=jax.ShapeDtypeStruct((NCORES, NPAD, CH), jnp.float32),
    mesh=_mesh(),
    compiler_params=pltpu.CompilerParams(needs_layout_passes=False),
    scratch_types=[
        pltpu.VMEM((HALF, CHUNK), jnp.int32),
        pltpu.VMEM((HALF, CHUNK), jnp.int32),
        pltpu.VMEM((CHUNK, CH), jnp.float32),
        pltpu.VMEM((CHUNK, CH), jnp.float32),
        pltpu.VMEM_SHARED((NPAD, CH), jnp.float32),
        pltpu.SemaphoreType.DMA,
        pltpu.SemaphoreType.DMA,
        pltpu.SemaphoreType.DMA,
        pltpu.SemaphoreType.DMA,
    ],
)


def _pre_body(x_ref, w_ref, deg_ref, y_ref, dis_ref):
    xw = jnp.dot(
        x_ref[...], w_ref[...],
        preferred_element_type=jnp.float32,
        precision=lax.Precision.HIGHEST,
    )
    dis = lax.rsqrt(deg_ref[...] + 1.0)
    y_ref[...] = xw * dis
    dis_ref[...] = dis


def _pre_call(x_pad, w, deg_rep):
    return pl.pallas_call(
        _pre_body,
        grid=(NPAD // BLK,),
        in_specs=[
            pl.BlockSpec((BLK, CH), lambda i: (i, 0)),
            pl.BlockSpec((CH, CH), lambda i: (0, 0)),
            pl.BlockSpec((BLK, CH), lambda i: (i, 0)),
        ],
        out_specs=[
            pl.BlockSpec((BLK, CH), lambda i: (i, 0)),
            pl.BlockSpec((BLK, CH), lambda i: (i, 0)),
        ],
        out_shape=[
            jax.ShapeDtypeStruct((NPAD, CH), jnp.float32),
            jax.ShapeDtypeStruct((NPAD, CH), jnp.float32),
        ],
    )(x_pad, w, deg_rep)


def _post_body(p_ref, y_ref, dis_ref, b_ref, a_ref, o_ref):
    si = p_ref[0, :, :] + p_ref[1, :, :] + y_ref[...]
    o = dis_ref[...] * si + b_ref[...]
    o_ref[...] = jnp.where(o > 0, o, a_ref[...] * o)


def _post_call(parts, y, dis, b2, a2):
    return pl.pallas_call(
        _post_body,
        grid=(NPAD // BLK,),
        in_specs=[
            pl.BlockSpec((NCORES, BLK, CH), lambda i: (0, i, 0)),
            pl.BlockSpec((BLK, CH), lambda i: (i, 0)),
            pl.BlockSpec((BLK, CH), lambda i: (i, 0)),
            pl.BlockSpec((1, CH), lambda i: (0, 0)),
            pl.BlockSpec((1, CH), lambda i: (0, 0)),
        ],
        out_specs=pl.BlockSpec((BLK, CH), lambda i: (i, 0)),
        out_shape=jax.ShapeDtypeStruct((NPAD, CH), jnp.float32),
    )(parts, y, dis, b2, a2)


def kernel(x, edge_index, W, b, prelu_a):
    row = edge_index[0]
    col = edge_index[1]
    e = row.shape[0]
    pad = EPAD - e
    rows = jnp.concatenate(
        [row, jnp.zeros((pad,), jnp.int32)]
    ).reshape(NTILES, NCHUNK, CHUNK)
    cols = jnp.concatenate(
        [col, jnp.full((pad,), DUMMY, jnp.int32)]
    ).reshape(NTILES, NCHUNK, CHUNK)
    x_pad = jnp.pad(x, ((0, NPAD - N), (0, 0)))

    deg_rep = _deg_call(cols.reshape(NSUB, CPT, CHUNK))
    y, dis = _pre_call(x_pad, W, deg_rep)
    parts = _msg_call(rows, cols, y)
    out = _post_call(parts, y, dis, b.reshape(1, CH), prelu_a.reshape(1, CH))
    return out[:N]
